# baseline (speedup 1.0000x reference)
import functools

import jax
import jax.numpy as jnp
from jax import lax
from jax.experimental import pallas as pl
from jax.experimental.pallas import tpu as pltpu
from jax.experimental.pallas import tpu_sc as plsc

N = 10000
F = 128
S = 16
OUT = 64
QUADS = S // 4

NC = 2
NS = 16
NW = NC * NS
VPW = 320
NPAD = NW * VPW
GV = 8
GROUP_ROWS = GV * S
NG = VPW // GV
ROW_W = 32

MM_BLOCK = 2000
NBUF = 4


def _pack_pair(y, h):
    u = jax.lax.bitcast_convert_type(
        y.astype(jnp.bfloat16), jnp.uint16).astype(jnp.uint32)
    lo = u[:, h * 64:h * 64 + 32]
    hi = u[:, h * 64 + 32:h * 64 + 64]
    return jax.lax.bitcast_convert_type((hi << 16) | lo, jnp.float32)


def _mm_body(x_ref, w_ref, o_ref):
    x = x_ref[...]
    for q in range(QUADS):
        parts = []
        for p in range(2):
            k = 2 * q + p
            y = jnp.dot(x, w_ref[:, k * F:(k + 1) * F],
                        preferred_element_type=jnp.float32)
            parts += [_pack_pair(y, 0), _pack_pair(y, 1)]
        o_ref[q] = jnp.concatenate(parts, axis=1)


def _project(x2d, wt):
    return pl.pallas_call(
        _mm_body,
        grid=(N // MM_BLOCK,),
        in_specs=[
            pl.BlockSpec((MM_BLOCK, F), lambda i: (i, 0)),
            pl.BlockSpec((F, S * OUT), lambda i: (0, 0)),
        ],
        out_specs=pl.BlockSpec((QUADS, MM_BLOCK, F), lambda i: (0, i, 0)),
        out_shape=jax.ShapeDtypeStruct((QUADS, N, F), jnp.float32),
    )(x2d, wt)


def _accum_group(buf, bias_v, out_v, g):
    for j in range(GV):
        r0 = j * S
        for w in range(2):
            pairs = [plsc.unpack(buf[r0 + s, pl.ds(w * 32, 32)],
                                 format=plsc.PackFormat.INTERLEAVED)
                     for s in range(S)]
            los = [p[0] for p in pairs]
            his = [p[1] for p in pairs]
            while len(los) > 1:
                los = [a + b for a, b in zip(los[::2], los[1::2])]
                his = [a + b for a, b in zip(his[::2], his[1::2])]
            alo = los[0] + bias_v[pl.ds(w * 16, 16)]
            ahi = his[0] + bias_v[pl.ds(32 + w * 16, 16)]
            alo = jnp.where(alo > 0.0, alo, jnp.exp(alo) - 1.0)
            ahi = jnp.where(ahi > 0.0, ahi, jnp.exp(ahi) - 1.0)
            out_v[g * GV + j, pl.ds(w * 16, 16)] = alo
            out_v[g * GV + j, pl.ds(32 + w * 16, 16)] = ahi


def _sc_body(table_hbm, idx_hbm, b_hbm, out_hbm,
             idx_v, buf0, buf1, buf2, buf3, out_v, bias_v,
             sem0, sem1, sem2, sem3):
    bufs = (buf0, buf1, buf2, buf3)
    sems = (sem0, sem1, sem2, sem3)
    wid = lax.axis_index("s") * NC + lax.axis_index("c")
    base_v = wid * VPW

    pltpu.sync_copy(b_hbm, bias_v)
    pltpu.sync_copy(idx_hbm.at[wid], idx_v)

    for b in range(NBUF):
        pltpu.async_copy(table_hbm.at[idx_v.at[b]], bufs[b], sems[b])

    def ring_body(t, carry):
        g0 = NBUF * t
        for b in range(NBUF):
            g = g0 + b
            pltpu.make_async_copy(
                table_hbm.at[idx_v.at[g]], bufs[b], sems[b]).wait()
            _accum_group(bufs[b], bias_v, out_v, g)

            @pl.when(g + NBUF < NG)
            def _():
                pltpu.async_copy(
                    table_hbm.at[idx_v.at[g + NBUF]], bufs[b], sems[b])
        return carry

    lax.fori_loop(0, NG // NBUF, ring_body, 0)
    pltpu.sync_copy(out_v, out_hbm.at[pl.ds(base_v, VPW)])


@functools.cache
def _sc_gather():
    return functools.partial(
        pl.kernel,
        mesh=plsc.VectorSubcoreMesh(core_axis_name="c", subcore_axis_name="s"),
        compiler_params=pltpu.CompilerParams(
            use_tc_tiling_on_sc=False, needs_layout_passes=False),
        out_type=jax.ShapeDtypeStruct((NPAD, OUT), jnp.float32),
        scratch_types=[
            pltpu.VMEM((NG, GROUP_ROWS), jnp.int32),
            pltpu.VMEM((GROUP_ROWS, 2 * ROW_W), jnp.bfloat16),
            pltpu.VMEM((GROUP_ROWS, 2 * ROW_W), jnp.bfloat16),
            pltpu.VMEM((GROUP_ROWS, 2 * ROW_W), jnp.bfloat16),
            pltpu.VMEM((GROUP_ROWS, 2 * ROW_W), jnp.bfloat16),
            pltpu.VMEM((VPW, OUT), jnp.float32),
            pltpu.VMEM((OUT,), jnp.float32),
            pltpu.SemaphoreType.DMA,
            pltpu.SemaphoreType.DMA,
            pltpu.SemaphoreType.DMA,
            pltpu.SemaphoreType.DMA,
        ],
    )(_sc_body)


def kernel(x, spiral_x, W, b):
    x2d = x.reshape(N, F)
    wt = W.reshape(OUT, S, F).transpose(2, 1, 0).reshape(F, S * OUT)
    table = _project(x2d, wt)

    ar = jnp.arange(S, dtype=jnp.int32)
    sidx = 4 * spiral_x[0] + (4 * N * (ar // 4) + ar % 4)[None, :]
    flat = (jnp.zeros((NPAD, S), jnp.int32).at[: N - 1].set(sidx)
            .reshape(NW, NG, GROUP_ROWS))

    table_bf = jax.lax.bitcast_convert_type(
        table, jnp.bfloat16).reshape(4 * QUADS * N, 2 * ROW_W)
    out = _sc_gather()(table_bf, flat, b)
    out = out[:N].at[N - 1].set(0.0)
    return out.reshape(1, N, OUT)


# speedup vs baseline: 42.8812x; 42.8812x over previous
import functools

import jax
import jax.numpy as jnp
from jax import lax
from jax.experimental import pallas as pl
from jax.experimental.pallas import tpu as pltpu
from jax.experimental.pallas import tpu_sc as plsc

N = 10000
F = 128
S = 16
OUT = 64
PAIRS = S // 2

NC = 2
NS = 16
NW = NC * NS
VPW = 320
NPAD = NW * VPW
GV = 8
GROUP_ROWS = GV * S
NG = VPW // GV

MM_BLOCK = 2000
NBUF = 4


def _mm_body(x_ref, w_ref, o_ref):
    x = x_ref[...]
    for k in range(PAIRS):
        o_ref[k] = jnp.dot(x, w_ref[:, k * F:(k + 1) * F],
                           preferred_element_type=jnp.float32)


def _project(x2d, wt):
    return pl.pallas_call(
        _mm_body,
        grid=(N // MM_BLOCK,),
        in_specs=[
            pl.BlockSpec((MM_BLOCK, F), lambda i: (i, 0)),
            pl.BlockSpec((F, S * OUT), lambda i: (0, 0)),
        ],
        out_specs=pl.BlockSpec((PAIRS, MM_BLOCK, F), lambda i: (0, i, 0)),
        out_shape=jax.ShapeDtypeStruct((PAIRS, N, F), jnp.float32),
    )(x2d, wt)


def _accum_group(buf, bias_v, out_v, g):
    for j in range(GV):
        r0 = j * S
        for c in range(OUT // 16):
            vals = [buf[r0 + s, pl.ds(c * 16, 16)] for s in range(S)]
            while len(vals) > 1:
                vals = [a + b for a, b in zip(vals[::2], vals[1::2])]
            acc = vals[0] + bias_v[pl.ds(c * 16, 16)]
            acc = jnp.where(acc > 0.0, acc, jnp.exp(acc) - 1.0)
            out_v[g * GV + j, pl.ds(c * 16, 16)] = acc


def _sc_body(table_hbm, idx_hbm, b_hbm, out_hbm,
             idx_v, buf0, buf1, buf2, buf3, out_v, bias_v,
             sem0, sem1, sem2, sem3):
    bufs = (buf0, buf1, buf2, buf3)
    sems = (sem0, sem1, sem2, sem3)
    wid = lax.axis_index("s") * NC + lax.axis_index("c")
    base_v = wid * VPW

    pltpu.sync_copy(b_hbm, bias_v)
    pltpu.sync_copy(idx_hbm.at[wid], idx_v)

    for b in range(NBUF):
        pltpu.async_copy(table_hbm.at[idx_v.at[b]], bufs[b], sems[b])

    def ring_body(t, carry):
        g0 = NBUF * t
        for b in range(NBUF):
            g = g0 + b
            pltpu.make_async_copy(
                table_hbm.at[idx_v.at[g]], bufs[b], sems[b]).wait()
            _accum_group(bufs[b], bias_v, out_v, g)

            @pl.when(g + NBUF < NG)
            def _():
                pltpu.async_copy(
                    table_hbm.at[idx_v.at[g + NBUF]], bufs[b], sems[b])
        return carry

    lax.fori_loop(0, NG // NBUF, ring_body, 0)
    pltpu.sync_copy(out_v, out_hbm.at[pl.ds(base_v, VPW)])


@functools.cache
def _sc_gather():
    return functools.partial(
        pl.kernel,
        mesh=plsc.VectorSubcoreMesh(core_axis_name="c", subcore_axis_name="s"),
        compiler_params=pltpu.CompilerParams(use_tc_tiling_on_sc=False),
        out_type=jax.ShapeDtypeStruct((NPAD, OUT), jnp.float32),
        scratch_types=[
            pltpu.VMEM((NG, GROUP_ROWS), jnp.int32),
            pltpu.VMEM((GROUP_ROWS, OUT), jnp.float32),
            pltpu.VMEM((GROUP_ROWS, OUT), jnp.float32),
            pltpu.VMEM((GROUP_ROWS, OUT), jnp.float32),
            pltpu.VMEM((GROUP_ROWS, OUT), jnp.float32),
            pltpu.VMEM((VPW, OUT), jnp.float32),
            pltpu.VMEM((OUT,), jnp.float32),
            pltpu.SemaphoreType.DMA,
            pltpu.SemaphoreType.DMA,
            pltpu.SemaphoreType.DMA,
            pltpu.SemaphoreType.DMA,
        ],
    )(_sc_body)


def kernel(x, spiral_x, W, b):
    x2d = x.reshape(N, F)
    wt = W.reshape(OUT, S, F).transpose(2, 1, 0).reshape(F, S * OUT)
    table = _project(x2d, wt)

    ar = jnp.arange(S, dtype=jnp.int32)
    sidx = 2 * spiral_x[0] + (2 * N * (ar // 2) + ar % 2)[None, :]
    flat = (jnp.zeros((NPAD, S), jnp.int32).at[: N - 1].set(sidx)
            .reshape(NW, NG, GROUP_ROWS))

    out = _sc_gather()(table.reshape(2 * PAIRS * N, OUT), flat, b)
    out = out[:N].at[N - 1].set(0.0)
    return out.reshape(1, N, OUT)
